# baseline (speedup 1.0000x reference)
import functools

import jax
import jax.numpy as jnp
from jax import lax
from jax.experimental import pallas as pl
from jax.experimental.pallas import tpu as pltpu, tpu_sc as plsc

N = 12800
B = 64
C = N // B
D = 64
V = 1000000
NW = 32
PER_W = N // NW
SEGS_W = B // NW
NG = PER_W // 16
GSEG = C // 16
GCH = 80
NCH = PER_W // GCH
BN = 65536
NB = -(-V // BN)
PR = NB * (BN // 32)

_mesh = plsc.VectorSubcoreMesh(core_axis_name="c", subcore_axis_name="s")


def _project_body(wt_ref, embt_ref, out_ref):
    pb = lax.dot_general(
        wt_ref[...], embt_ref[...], (((1,), (0,)), ((), ())),
        preferred_element_type=jnp.float32,
        precision=lax.Precision.DEFAULT,
    )
    t = pb[0:4].reshape(4, BN // 32, 32)
    out_ref[...] = jnp.transpose(t, (1, 0, 2)).reshape(BN // 32, 128)


_project = pl.pallas_call(
    _project_body,
    grid=(NB,),
    in_specs=[
        pl.BlockSpec((8, D), lambda k: (0, 0)),
        pl.BlockSpec((D, BN), lambda k: (0, k)),
    ],
    out_specs=pl.BlockSpec((BN // 32, 128), lambda k: (k, 0)),
    out_shape=jax.ShapeDtypeStruct((PR, 128), jnp.float32),
)


@functools.partial(
    pl.kernel,
    out_type=jax.ShapeDtypeStruct((B * 16,), jnp.float32),
    mesh=_mesh,
    compiler_params=pltpu.CompilerParams(
        use_tc_tiling_on_sc=False, needs_layout_passes=False),
    scratch_types=[
        pltpu.VMEM((PER_W,), jnp.int32),
        pltpu.VMEM((PER_W,), jnp.int32),
        pltpu.VMEM((PER_W, 128), jnp.float32),
        pltpu.VMEM((16,), jnp.float32),
        pltpu.VMEM((16,), jnp.float32),
        pltpu.SemaphoreType.DMA,
    ],
)
def _sc_pool(text_hbm, pp_hbm, b_hbm, out_hbm,
             idx_v, ridx_v, rows_v, b_v, out_v, sem):
    wid = lax.axis_index("s") * 2 + lax.axis_index("c")
    base = wid * PER_W

    pltpu.sync_copy(text_hbm.at[pl.ds(base, PER_W)], idx_v)
    pltpu.sync_copy(b_hbm, b_v)

    def rid_body(g, carry):
        iv = idx_v[pl.ds(g * 16, 16)]
        ridx_v[pl.ds(g * 16, 16)] = lax.shift_right_logical(iv, 5)
        return carry
    lax.fori_loop(0, NG, rid_body, 0)

    copies = [
        pltpu.async_copy(
            pp_hbm.at[ridx_v.at[pl.ds(k * GCH, GCH)]],
            rows_v.at[pl.ds(k * GCH, GCH)],
            sem,
        )
        for k in range(NCH)
    ]
    for cp in copies:
        cp.wait()

    lane = lax.iota(jnp.int32, 16)
    zero = jnp.zeros((16,), jnp.float32)

    def grp_vals(g):
        iv = idx_v[pl.ds(g * 16, 16)]
        vm = jnp.bitwise_and(iv, 31)
        i_vec = g * 16 + lane
        return [plsc.load_gather(rows_v, [i_vec, j * 32 + vm])
                for j in range(4)]

    def acc_body(g, vjs):
        vals = grp_vals(g)
        return tuple(vjs[j] + vals[j] for j in range(4))

    vjs0 = lax.fori_loop(0, GSEG, acc_body, (zero, zero, zero, zero))
    vjs1 = lax.fori_loop(GSEG + 1, NG, acc_body, (zero, zero, zero, zero))

    vals = grp_vals(GSEG)
    lo = lane < 8
    vjs0 = tuple(vjs0[j] + jnp.where(lo, vals[j], 0.0) for j in range(4))
    vjs1 = tuple(vjs1[j] + jnp.where(lo, 0.0, vals[j]) for j in range(4))

    for s, vjs in ((0, vjs0), (1, vjs1)):
        out = b_v[pl.ds(0, 16)]
        for j in range(4):
            r = vjs[j]
            tot = r[0]
            for l in range(1, 16):
                tot = tot + r[l]
            out = out + jnp.where(lane == j, tot, 0.0)
        out_v[pl.ds(0, 16)] = out
        pltpu.sync_copy(out_v, out_hbm.at[pl.ds((wid * SEGS_W + s) * 16, 16)])


def kernel(text, emb, W, b):
    text = text.astype(jnp.int32)
    wt8 = jnp.zeros((8, D), jnp.float32).at[:4].set(W.astype(jnp.float32) / C)
    b16 = jnp.zeros((16,), jnp.float32).at[:4].set(b.astype(jnp.float32))
    pp = _project(wt8, emb.T)
    flat = _sc_pool(text, pp, b16)
    return flat.reshape(B, 16)[:, :4]


# speedup vs baseline: 1.0270x; 1.0270x over previous
import functools

import jax
import jax.numpy as jnp
from jax import lax
from jax.experimental import pallas as pl
from jax.experimental.pallas import tpu as pltpu, tpu_sc as plsc

N = 12800
B = 64
C = N // B
D = 64
V = 1000000
NW = 32
PER_W = N // NW
SEGS_W = B // NW
NG = PER_W // 16
GSEG = C // 16
GCH = 80
NCH = PER_W // GCH
BN = 65536
NB = -(-V // BN)
PR = NB * (BN // 32)

_mesh = plsc.VectorSubcoreMesh(core_axis_name="c", subcore_axis_name="s")


def _project_body(w_ref, embt_ref, out_ref):
    pb = lax.dot_general(
        w_ref[...] * (1.0 / C), embt_ref[...], (((1,), (0,)), ((), ())),
        preferred_element_type=jnp.float32,
        precision=lax.Precision.DEFAULT,
    )
    t = pb.reshape(4, BN // 32, 32)
    out_ref[...] = jnp.transpose(t, (1, 0, 2)).reshape(BN // 32, 128)


_project = pl.pallas_call(
    _project_body,
    grid=(NB,),
    in_specs=[
        pl.BlockSpec((4, D), lambda k: (0, 0)),
        pl.BlockSpec((D, BN), lambda k: (0, k)),
    ],
    out_specs=pl.BlockSpec((BN // 32, 128), lambda k: (k, 0)),
    out_shape=jax.ShapeDtypeStruct((PR, 128), jnp.float32),
)


@functools.partial(
    pl.kernel,
    out_type=jax.ShapeDtypeStruct((B * 16,), jnp.float32),
    mesh=_mesh,
    compiler_params=pltpu.CompilerParams(
        use_tc_tiling_on_sc=False, needs_layout_passes=False),
    scratch_types=[
        pltpu.VMEM((PER_W,), jnp.int32),
        pltpu.VMEM((PER_W,), jnp.int32),
        pltpu.VMEM((PER_W, 128), jnp.float32),
        pltpu.VMEM((16,), jnp.float32),
        pltpu.SemaphoreType.DMA,
    ],
)
def _sc_pool(text_hbm, pp_hbm, out_hbm,
             idx_v, ridx_v, rows_v, out_v, sem):
    wid = lax.axis_index("s") * 2 + lax.axis_index("c")
    base = wid * PER_W

    pltpu.sync_copy(text_hbm.at[pl.ds(base, PER_W)], idx_v)

    def rid_body(g, carry):
        iv = idx_v[pl.ds(g * 16, 16)]
        ridx_v[pl.ds(g * 16, 16)] = lax.shift_right_logical(iv, 5)
        return carry
    lax.fori_loop(0, NG, rid_body, 0)

    copies = [
        pltpu.async_copy(
            pp_hbm.at[ridx_v.at[pl.ds(k * GCH, GCH)]],
            rows_v.at[pl.ds(k * GCH, GCH)],
            sem,
        )
        for k in range(NCH)
    ]
    for cp in copies:
        cp.wait()

    lane = lax.iota(jnp.int32, 16)
    zero = jnp.zeros((16,), jnp.float32)

    def grp_vals(g):
        iv = idx_v[pl.ds(g * 16, 16)]
        vm = jnp.bitwise_and(iv, 31)
        i_vec = g * 16 + lane
        return [plsc.load_gather(rows_v, [i_vec, j * 32 + vm])
                for j in range(4)]

    def acc_body(g, vjs):
        vals = grp_vals(g)
        return tuple(vjs[j] + vals[j] for j in range(4))

    vjs0 = lax.fori_loop(0, GSEG, acc_body, (zero, zero, zero, zero))
    vjs1 = lax.fori_loop(GSEG + 1, NG, acc_body, (zero, zero, zero, zero))

    vals = grp_vals(GSEG)
    lo = lane < 8
    vjs0 = tuple(vjs0[j] + jnp.where(lo, vals[j], 0.0) for j in range(4))
    vjs1 = tuple(vjs1[j] + jnp.where(lo, 0.0, vals[j]) for j in range(4))

    for s, vjs in ((0, vjs0), (1, vjs1)):
        out = zero
        for j in range(4):
            r = vjs[j]
            tot = r[0]
            for l in range(1, 16):
                tot = tot + r[l]
            out = out + jnp.where(lane == j, tot, 0.0)
        out_v[pl.ds(0, 16)] = out
        pltpu.sync_copy(out_v, out_hbm.at[pl.ds((wid * SEGS_W + s) * 16, 16)])


def kernel(text, emb, W, b):
    text = text.astype(jnp.int32)
    pp = _project(W.astype(jnp.float32), emb.T)
    flat = _sc_pool(text, pp)
    return flat.reshape(B, 16)[:, :4] + b.astype(jnp.float32)


# speedup vs baseline: 1.0279x; 1.0009x over previous
import functools

import jax
import jax.numpy as jnp
from jax import lax
from jax.experimental import pallas as pl
from jax.experimental.pallas import tpu as pltpu, tpu_sc as plsc

N = 12800
B = 64
C = N // B
D = 64
V = 1000000
NW = 32
PER_W = N // NW
SEGS_W = B // NW
NG = PER_W // 16
GSEG = C // 16
GCH = 80
NCH = PER_W // GCH
BN = 65536
NB = -(-V // BN)
PR = NB * (BN // 32)

_mesh = plsc.VectorSubcoreMesh(core_axis_name="c", subcore_axis_name="s")


def _project_body(w_ref, embt_ref, out_ref):
    pb = lax.dot_general(
        w_ref[...] * (1.0 / C), embt_ref[...], (((1,), (0,)), ((), ())),
        preferred_element_type=jnp.float32,
        precision=lax.Precision.DEFAULT,
    )
    t = pb.reshape(4, BN // 32, 32)
    out_ref[...] = jnp.transpose(t, (1, 0, 2)).reshape(BN // 32, 128)


_project = pl.pallas_call(
    _project_body,
    grid=(NB,),
    in_specs=[
        pl.BlockSpec((4, D), lambda k: (0, 0)),
        pl.BlockSpec((D, BN), lambda k: (0, k)),
    ],
    out_specs=pl.BlockSpec((BN // 32, 128), lambda k: (k, 0)),
    out_shape=jax.ShapeDtypeStruct((PR, 128), jnp.float32),
)


@functools.partial(
    pl.kernel,
    out_type=jax.ShapeDtypeStruct((B * 16,), jnp.float32),
    mesh=_mesh,
    compiler_params=pltpu.CompilerParams(
        use_tc_tiling_on_sc=False, needs_layout_passes=False),
    scratch_types=[
        pltpu.VMEM((PER_W,), jnp.int32),
        pltpu.VMEM((PER_W,), jnp.int32),
        pltpu.VMEM((PER_W, 128), jnp.float32),
        pltpu.VMEM((16,), jnp.float32),
        pltpu.SemaphoreType.DMA,
    ],
)
def _sc_pool(text_hbm, pp_hbm, out_hbm,
             idx_v, ridx_v, rows_v, out_v, sem):
    wid = lax.axis_index("s") * 2 + lax.axis_index("c")
    base = wid * PER_W

    pltpu.sync_copy(text_hbm.at[pl.ds(base, PER_W)], idx_v)

    def rid_body(g, carry):
        iv = idx_v[pl.ds(g * 16, 16)]
        ridx_v[pl.ds(g * 16, 16)] = lax.shift_right_logical(iv, 5)
        return carry
    lax.fori_loop(0, NG, rid_body, 0)

    copies = [
        pltpu.async_copy(
            pp_hbm.at[ridx_v.at[pl.ds(k * GCH, GCH)]],
            rows_v.at[pl.ds(k * GCH, GCH)],
            sem,
        )
        for k in range(NCH)
    ]

    lane = lax.iota(jnp.int32, 16)
    zero = jnp.zeros((16,), jnp.float32)
    lo = lane < 8

    def grp_vals(g):
        iv = idx_v[pl.ds(g * 16, 16)]
        vm = jnp.bitwise_and(iv, 31)
        i_vec = g * 16 + lane
        return [plsc.load_gather(rows_v, [i_vec, j * 32 + vm])
                for j in range(4)]

    vjs0 = [zero] * 4
    vjs1 = [zero] * 4
    gpc = GCH // 16
    for k in range(NCH):
        copies[k].wait()
        for g in range(k * gpc, (k + 1) * gpc):
            vals = grp_vals(g)
            if g < GSEG:
                vjs0 = [vjs0[j] + vals[j] for j in range(4)]
            elif g > GSEG:
                vjs1 = [vjs1[j] + vals[j] for j in range(4)]
            else:
                vjs0 = [vjs0[j] + jnp.where(lo, vals[j], 0.0)
                        for j in range(4)]
                vjs1 = [vjs1[j] + jnp.where(lo, 0.0, vals[j])
                        for j in range(4)]

    for s, vjs in ((0, vjs0), (1, vjs1)):
        out = zero
        for j in range(4):
            r = vjs[j]
            tot = r[0]
            for l in range(1, 16):
                tot = tot + r[l]
            out = out + jnp.where(lane == j, tot, 0.0)
        out_v[pl.ds(0, 16)] = out
        pltpu.sync_copy(out_v, out_hbm.at[pl.ds((wid * SEGS_W + s) * 16, 16)])


def kernel(text, emb, W, b):
    text = text.astype(jnp.int32)
    pp = _project(W.astype(jnp.float32), emb.T)
    flat = _sc_pool(text, pp)
    return flat.reshape(B, 16)[:, :4] + b.astype(jnp.float32)


# speedup vs baseline: 1.0660x; 1.0370x over previous
import functools

import jax
import jax.numpy as jnp
from jax import lax
from jax.experimental import pallas as pl
from jax.experimental.pallas import tpu as pltpu, tpu_sc as plsc

N = 12800
B = 64
C = N // B
D = 64
V = 1000000
NW = 32
PER_W = N // NW
SEGS_W = B // NW
NG = PER_W // 16
GSEG = C // 16
GCH = 80
NCH = PER_W // GCH
BN = 65536
NB = -(-V // BN)
PR = NB * (BN // 64)

_mesh = plsc.VectorSubcoreMesh(core_axis_name="c", subcore_axis_name="s")


def _project_body(w_ref, embt_ref, out_ref):
    pb = lax.dot_general(
        w_ref[...] * (1.0 / C), embt_ref[...], (((1,), (0,)), ((), ())),
        preferred_element_type=jnp.float32,
        precision=lax.Precision.DEFAULT,
    )
    u = lax.bitcast_convert_type(pb.astype(jnp.bfloat16), jnp.uint16)
    u = u.astype(jnp.int32)
    w01 = jnp.bitwise_or(u[0], jnp.left_shift(u[1], 16))
    w23 = jnp.bitwise_or(u[2], jnp.left_shift(u[3], 16))
    t = jnp.stack([w01, w23]).reshape(2, BN // 64, 64)
    out_ref[...] = jnp.transpose(t, (1, 0, 2)).reshape(BN // 64, 128)


_project = pl.pallas_call(
    _project_body,
    grid=(NB,),
    in_specs=[
        pl.BlockSpec((4, D), lambda k: (0, 0)),
        pl.BlockSpec((D, BN), lambda k: (0, k)),
    ],
    out_specs=pl.BlockSpec((BN // 64, 128), lambda k: (k, 0)),
    out_shape=jax.ShapeDtypeStruct((PR, 128), jnp.int32),
)


@functools.partial(
    pl.kernel,
    out_type=jax.ShapeDtypeStruct((B * 16,), jnp.float32),
    mesh=_mesh,
    compiler_params=pltpu.CompilerParams(
        use_tc_tiling_on_sc=False, needs_layout_passes=False),
    scratch_types=[
        pltpu.VMEM((PER_W,), jnp.int32),
        pltpu.VMEM((PER_W,), jnp.int32),
        pltpu.VMEM((PER_W, 128), jnp.int32),
        pltpu.VMEM((16,), jnp.float32),
        pltpu.SemaphoreType.DMA,
    ],
)
def _sc_pool(text_hbm, pp_hbm, out_hbm,
             idx_v, ridx_v, rows_v, out_v, sem):
    wid = lax.axis_index("s") * 2 + lax.axis_index("c")
    base = wid * PER_W

    pltpu.sync_copy(text_hbm.at[pl.ds(base, PER_W)], idx_v)

    def rid_body(g, carry):
        iv = idx_v[pl.ds(g * 16, 16)]
        ridx_v[pl.ds(g * 16, 16)] = lax.shift_right_logical(iv, 6)
        return carry
    lax.fori_loop(0, NG, rid_body, 0)

    copies = [
        pltpu.async_copy(
            pp_hbm.at[ridx_v.at[pl.ds(k * GCH, GCH)]],
            rows_v.at[pl.ds(k * GCH, GCH)],
            sem,
        )
        for k in range(NCH)
    ]

    lane = lax.iota(jnp.int32, 16)
    zero = jnp.zeros((16,), jnp.float32)
    lo = lane < 8

    himask = jnp.full((16,), -65536, jnp.int32)

    def grp_vals(g):
        iv = idx_v[pl.ds(g * 16, 16)]
        vm = jnp.bitwise_and(iv, 63)
        i_vec = g * 16 + lane
        w01 = plsc.load_gather(rows_v, [i_vec, vm])
        w23 = plsc.load_gather(rows_v, [i_vec, 64 + vm])
        return [plsc.bitcast(lax.shift_left(w01, 16), jnp.float32),
                plsc.bitcast(jnp.bitwise_and(w01, himask), jnp.float32),
                plsc.bitcast(lax.shift_left(w23, 16), jnp.float32),
                plsc.bitcast(jnp.bitwise_and(w23, himask), jnp.float32)]

    vjs0 = [zero] * 4
    vjs1 = [zero] * 4
    gpc = GCH // 16
    for k in range(NCH):
        copies[k].wait()
        for g in range(k * gpc, (k + 1) * gpc):
            vals = grp_vals(g)
            if g < GSEG:
                vjs0 = [vjs0[j] + vals[j] for j in range(4)]
            elif g > GSEG:
                vjs1 = [vjs1[j] + vals[j] for j in range(4)]
            else:
                vjs0 = [vjs0[j] + jnp.where(lo, vals[j], 0.0)
                        for j in range(4)]
                vjs1 = [vjs1[j] + jnp.where(lo, 0.0, vals[j])
                        for j in range(4)]

    for s, vjs in ((0, vjs0), (1, vjs1)):
        out = zero
        for j in range(4):
            r = vjs[j]
            tot = r[0]
            for l in range(1, 16):
                tot = tot + r[l]
            out = out + jnp.where(lane == j, tot, 0.0)
        out_v[pl.ds(0, 16)] = out
        pltpu.sync_copy(out_v, out_hbm.at[pl.ds((wid * SEGS_W + s) * 16, 16)])


def kernel(text, emb, W, b):
    text = text.astype(jnp.int32)
    pp = _project(W.astype(jnp.float32), emb.T)
    flat = _sc_pool(text, pp)
    return flat.reshape(B, 16)[:, :4] + b.astype(jnp.float32)
